# Initial kernel scaffold; baseline (speedup 1.0000x reference)
import jax
import jax.numpy as jnp
from jax.experimental import pallas as pl


def kernel(x, positions, c, time_emb, Wq, bq, Wk, bk, Wv, bv, Wo, bo, W1, b1f, W2, b2f, g1, be1, g2, be2, log_tau):
    raise NotImplementedError("write your pallas kernel here")


# speedup vs baseline: 13.0057x; 13.0057x over previous
import functools
import math

import jax
import jax.numpy as jnp
from jax.experimental import pallas as pl

EPS = 1e-08
DIM = 256
HEADS = 8
TOPK = 32
HD = DIM // HEADS
BN = 256


def _ln(x, g, b):
    m = jnp.mean(x, axis=-1, keepdims=True)
    v = jnp.mean((x - m) ** 2, axis=-1, keepdims=True)
    return (x - m) / jnp.sqrt(v + 1e-5) * g + b


def _qkv_kernel(x_ref, te_ref, g1_ref, be1_ref,
                wq_ref, bq_ref, wk_ref, bk_ref, wv_ref, bv_ref,
                q_ref, k_ref, v_ref):
    x = x_ref[0]
    xn1 = _ln(x, g1_ref[0], be1_ref[0]) + te_ref[0]
    dn = (((1,), (1,)), ((), ()))
    q_ref[0] = jax.lax.dot_general(xn1, wq_ref[...], dn,
                                   preferred_element_type=jnp.float32) + bq_ref[0]
    k_ref[0] = jax.lax.dot_general(xn1, wk_ref[...], dn,
                                   preferred_element_type=jnp.float32) + bk_ref[0]
    v_ref[0] = jax.lax.dot_general(xn1, wv_ref[...], dn,
                                   preferred_element_type=jnp.float32) + bv_ref[0]


def _attn_kernel(posi_ref, posj_ref, c_ref, lt_ref,
                 q_ref, k_ref, v_ref, x_ref,
                 wo_ref, bo_ref, g2_ref, be2_ref,
                 w1_ref, b1_ref, w2_ref, b2_ref,
                 out_ref):
    n = posj_ref.shape[2]
    c = c_ref[0, 0]

    pos_i = posi_ref[0]
    pos_j = posj_ref[0]
    xi = pos_i[:, 0:1]
    yi = pos_i[:, 1:2]
    xj = pos_j[0:1, :]
    yj = pos_j[1:2, :]
    dx = xi - xj
    dy = yi - yj
    diff = dx * dx + dy * dy
    ni = xi * xi + yi * yi
    nj = xj * xj + yj * yj
    den = (1.0 - c * nj) * (1.0 - c * ni)
    arg = jnp.maximum(1.0 + (2.0 * c * diff) / (den + EPS), 1.0)

    bits = jax.lax.bitcast_convert_type(arg, jnp.int32)
    lo = jnp.full((BN, 1), jnp.int32(0x3F800000))
    hi = jnp.max(bits, axis=1, keepdims=True)

    def body(_, carry):
        lo, hi = carry
        mid = lo + jax.lax.shift_right_logical(hi - lo, 1)
        cnt = jnp.sum((bits <= mid).astype(jnp.int32), axis=1, keepdims=True)
        take = cnt >= TOPK
        return jnp.where(take, lo, mid + 1), jnp.where(take, mid, hi)

    lo, hi = jax.lax.fori_loop(0, 31, body, (lo, hi))
    mask = bits <= hi

    acosh = jnp.log(arg + jnp.sqrt((arg - 1.0) * (arg + 1.0)))
    dist = acosh * jax.lax.rsqrt(c)
    tau = jnp.exp(lt_ref[0, 0]) + EPS
    base = jnp.where(mask, dist * (-1.0 / tau), -1e30)

    q = q_ref[0]
    k = k_ref[0]
    v = v_ref[0]
    scale = 1.0 / math.sqrt(HD)
    dn_nt = (((1,), (1,)), ((), ()))
    dn_nn = (((1,), (0,)), ((), ()))
    heads = []
    for h in range(HEADS):
        sl = slice(h * HD, (h + 1) * HD)
        scores = jax.lax.dot_general(q[:, sl], k[:, sl], dn_nt,
                                     preferred_element_type=jnp.float32)
        comb = scores * scale + base
        m = jnp.max(comb, axis=1, keepdims=True)
        e = jnp.exp(comb - m)
        p = e / jnp.sum(e, axis=1, keepdims=True)
        heads.append(jax.lax.dot_general(p, v[:, sl], dn_nn,
                                         preferred_element_type=jnp.float32))
    ao = jnp.concatenate(heads, axis=1)

    x2 = x_ref[0] + jax.lax.dot_general(ao, wo_ref[...], dn_nt,
                                        preferred_element_type=jnp.float32) + bo_ref[0]
    h2 = _ln(x2, g2_ref[0], be2_ref[0])
    t1 = jax.lax.dot_general(h2, w1_ref[...], dn_nt,
                             preferred_element_type=jnp.float32) + b1_ref[0]
    gl = 0.5 * t1 * (1.0 + jax.lax.erf(t1 * (1.0 / math.sqrt(2.0))))
    ffn = jax.lax.dot_general(gl, w2_ref[...], dn_nt,
                              preferred_element_type=jnp.float32) + b2_ref[0]
    out_ref[0] = x2 + ffn


@jax.jit
def kernel(x, positions, c, time_emb, Wq, bq, Wk, bk, Wv, bv, Wo, bo,
           W1, b1f, W2, b2f, g1, be1, g2, be2, log_tau):
    B, N, D = x.shape
    nb = N // BN
    r1 = lambda a: a.reshape(1, -1)
    c2 = c.reshape(1, 1)
    lt2 = log_tau.reshape(1, 1)
    pos_t = jnp.transpose(positions, (0, 2, 1))

    row_blk = pl.BlockSpec((1, BN, D), lambda b, r: (b, r, 0))
    full_kv = pl.BlockSpec((1, N, D), lambda b, r: (b, 0, 0))
    wfull = lambda a: pl.BlockSpec(a.shape, lambda b, r: (0,) * a.ndim)

    q, k, v = pl.pallas_call(
        _qkv_kernel,
        grid=(B, nb),
        in_specs=[
            row_blk,
            pl.BlockSpec((1, 1, D), lambda b, r: (b, 0, 0)),
            wfull(r1(g1)), wfull(r1(be1)),
            wfull(Wq), wfull(r1(bq)),
            wfull(Wk), wfull(r1(bk)),
            wfull(Wv), wfull(r1(bv)),
        ],
        out_specs=[row_blk, row_blk, row_blk],
        out_shape=[jax.ShapeDtypeStruct((B, N, D), jnp.float32)] * 3,
    )(x, time_emb.reshape(B, 1, D), r1(g1), r1(be1),
      Wq, r1(bq), Wk, r1(bk), Wv, r1(bv))

    out = pl.pallas_call(
        _attn_kernel,
        grid=(B, nb),
        in_specs=[
            pl.BlockSpec((1, BN, 2), lambda b, r: (b, r, 0)),
            pl.BlockSpec((1, 2, N), lambda b, r: (b, 0, 0)),
            wfull(c2), wfull(lt2),
            row_blk, full_kv, full_kv, row_blk,
            wfull(Wo), wfull(r1(bo)),
            wfull(r1(g2)), wfull(r1(be2)),
            wfull(W1), wfull(r1(b1f)),
            wfull(W2), wfull(r1(b2f)),
        ],
        out_specs=row_blk,
        out_shape=jax.ShapeDtypeStruct((B, N, D), jnp.float32),
    )(positions, pos_t, c2, lt2, q, k, v, x,
      Wo, r1(bo), r1(g2), r1(be2), W1, r1(b1f), W2, r1(b2f))
    return out
